# baseline (speedup 1.0000x reference)
import functools

import jax
import jax.numpy as jnp
from jax import lax
from jax.experimental import pallas as pl
from jax.experimental.pallas import tpu as pltpu
from jax.experimental.pallas import tpu_sc as plsc

ALPHA = 0.5
N = 10000
E = 160000
D = 256
DC = 128
NSUB = 16
NPAD = 10240
STRIPE = NPAD // NSUB
B = 128
NBS = -(--(-E // (NSUB * B)) // 8) * 8
ROWS = NSUB * NBS
EPAD = ROWS * B
CW = 16
NB = 5
IB = 16
BLK = 1000


def _sc_body(src_ref, dst_ref, x0, x1,
             s1_ref, s2_ref, cnt_ref,
             acc, cntacc, gidxs, sidxs, r0, r1, r2, r3, r4, onesb, zbc,
             gsems, ssems, csem):
    c = lax.axis_index("c")
    s = lax.axis_index("s")
    base = s * NBS
    st = s * STRIPE
    rows = (r0, r1, r2, r3, r4)

    def fill(j, carry):
        onesb[j, pl.ds(0, CW)] = jnp.ones((CW,), jnp.float32)
        zbc[j, pl.ds(0, CW)] = jnp.zeros((CW,), jnp.float32)
        return carry
    lax.fori_loop(0, B, fill, 0)

    def gwait(k):
        pltpu.make_async_copy(x0.at[pl.ds(0, B)], rows[k], gsems.at[k]).wait()

    def swait(k):
        pltpu.make_async_copy(rows[k], acc.at[pl.ds(0, B)], ssems.at[k]).wait()

    def cwait():
        pltpu.make_async_copy(onesb, cntacc.at[pl.ds(0, B)], csem).wait()

    def gissue(j, k):
        @pl.when(c == 0)
        def _():
            pltpu.async_copy(x0.at[gidxs.at[j]], rows[k], gsems.at[k])

        @pl.when(c == 1)
        def _():
            pltpu.async_copy(x1.at[gidxs.at[j]], rows[k], gsems.at[k])

    def run_pass(gat_hbm, sca_hbm, out_ref, pass_id):
        cntp = c == pass_id

        def zfill(j, carry):
            for kk in range(DC // 32):
                r0[j, pl.ds(32 * kk, 32)] = jnp.zeros((32,), jnp.bfloat16)
            return carry
        lax.fori_loop(0, B, zfill, 0)
        for r in range(STRIPE // B):
            pltpu.sync_copy(r0, acc.at[pl.ds(st + r * B, B)])

        @pl.when(cntp)
        def _():
            for r in range(STRIPE // B):
                pltpu.sync_copy(zbc, cntacc.at[pl.ds(st + r * B, B)])

        pltpu.sync_copy(gat_hbm.at[pl.ds(base, NBS)], gidxs)
        pltpu.sync_copy(sca_hbm.at[pl.ds(base, NBS)], sidxs)
        plsc.subcore_barrier()

        for k in range(3):
            gissue(k, k)

        def group(q, carry):
            for k in range(NB):
                j = NB * q + k
                pk = (k + 3) % NB

                @pl.when(j >= 2)
                def _():
                    swait(pk)

                @pl.when(j + 3 < NBS)
                def _():
                    gissue(j + 3, pk)

                gwait(k)
                pltpu.async_copy(rows[k], acc.at[sidxs.at[j]], ssems.at[k],
                                 add=True)

                @pl.when(cntp & (j >= 1))
                def _():
                    cwait()

                @pl.when(cntp)
                def _():
                    pltpu.async_copy(onesb, cntacc.at[sidxs.at[j]], csem,
                                     add=True)
            return carry

        lax.fori_loop(0, NBS // NB, group, 0)
        swait((NBS - 2) % NB)
        swait((NBS - 1) % NB)

        @pl.when(cntp)
        def _():
            cwait()

        plsc.subcore_barrier()

        @pl.when(c == 0)
        def _():
            pltpu.sync_copy(acc.at[pl.ds(st, STRIPE)],
                            out_ref.at[0, pl.ds(st, STRIPE)])

        @pl.when(c == 1)
        def _():
            pltpu.sync_copy(acc.at[pl.ds(st, STRIPE)],
                            out_ref.at[1, pl.ds(st, STRIPE)])

        @pl.when(cntp)
        def _():
            pltpu.sync_copy(cntacc.at[pl.ds(st, STRIPE)],
                            cnt_ref.at[pass_id, pl.ds(st, STRIPE)])

        plsc.subcore_barrier()

    run_pass(src_ref, dst_ref, s1_ref, 0)
    run_pass(dst_ref, src_ref, s2_ref, 1)


_sc_agg = pl.kernel(
    _sc_body,
    out_type=(
        jax.ShapeDtypeStruct((2, NPAD, DC), jnp.bfloat16),
        jax.ShapeDtypeStruct((2, NPAD, DC), jnp.bfloat16),
        jax.ShapeDtypeStruct((2, NPAD, CW), jnp.float32),
    ),
    mesh=plsc.VectorSubcoreMesh(core_axis_name="c", subcore_axis_name="s"),
    compiler_params=pltpu.CompilerParams(use_tc_tiling_on_sc=False),
    scratch_types=[
        pltpu.VMEM_SHARED((NPAD, DC), jnp.bfloat16),
        pltpu.VMEM_SHARED((NPAD, CW), jnp.float32),
        pltpu.VMEM((NBS, B), jnp.int32),
        pltpu.VMEM((NBS, B), jnp.int32),
        pltpu.VMEM((B, DC), jnp.bfloat16),
        pltpu.VMEM((B, DC), jnp.bfloat16),
        pltpu.VMEM((B, DC), jnp.bfloat16),
        pltpu.VMEM((B, DC), jnp.bfloat16),
        pltpu.VMEM((B, DC), jnp.bfloat16),
        pltpu.VMEM((B, CW), jnp.float32),
        pltpu.VMEM((B, CW), jnp.float32),
        pltpu.SemaphoreType.DMA((NB,)),
        pltpu.SemaphoreType.DMA((NB,)),
        pltpu.SemaphoreType.DMA,
    ],
)


def _self_body(x0_ref, x1_ref, wst_ref, b_ref, o_ref):
    out = b_ref[...]
    for t in range(2):
        out += jnp.dot((x0_ref, x1_ref)[t][...], wst_ref[t * DC:(t + 1) * DC, :],
                       preferred_element_type=jnp.float32)
    o_ref[...] = out


def _self_mm(x0, x1, wst, bias):
    return pl.pallas_call(
        _self_body,
        grid=(N // BLK,),
        in_specs=[
            pl.BlockSpec((BLK, DC), lambda i: (i, 0)),
            pl.BlockSpec((BLK, DC), lambda i: (i, 0)),
            pl.BlockSpec((D, D), lambda i: (0, 0)),
            pl.BlockSpec((1, D), lambda i: (0, 0)),
        ],
        out_specs=pl.BlockSpec((BLK, D), lambda i: (i, 0)),
        out_shape=jax.ShapeDtypeStruct((N, D), jnp.float32),
    )(x0, x1, wst, bias)


def _combine_body(self_ref, s1_ref, s2_ref, cnt_ref, w1t_ref, w2t_ref, o_ref):
    r1 = 1.0 / jnp.maximum(cnt_ref[0, :, 0:1], 1.0)
    r2 = 1.0 / jnp.maximum(cnt_ref[1, :, 0:1], 1.0)
    m1 = jnp.zeros_like(self_ref[...])
    m2 = jnp.zeros_like(m1)
    for t in range(2):
        w = slice(t * DC, (t + 1) * DC)
        m1 += jnp.dot(s1_ref[t], w1t_ref[w, :],
                      preferred_element_type=jnp.float32)
        m2 += jnp.dot(s2_ref[t], w2t_ref[w, :],
                      preferred_element_type=jnp.float32)
    o_ref[...] = self_ref[...] + (1.0 - ALPHA) * r1 * m1 + ALPHA * r2 * m2


def _combine(self_out, s1, s2, cnt, w1t, w2t):
    return pl.pallas_call(
        _combine_body,
        grid=(N // BLK,),
        in_specs=[
            pl.BlockSpec((BLK, D), lambda i: (i, 0)),
            pl.BlockSpec((2, BLK, DC), lambda i: (0, i, 0)),
            pl.BlockSpec((2, BLK, DC), lambda i: (0, i, 0)),
            pl.BlockSpec((2, BLK, CW), lambda i: (0, i, 0)),
            pl.BlockSpec((D, D), lambda i: (0, 0)),
            pl.BlockSpec((D, D), lambda i: (0, 0)),
        ],
        out_specs=pl.BlockSpec((BLK, D), lambda i: (i, 0)),
        out_shape=jax.ShapeDtypeStruct((N, D), jnp.float32),
    )(self_out, s1, s2, cnt, w1t, w2t)


def kernel(x, edge_index, W_self, b_self, W1, b1, W2, b2):
    src = edge_index[0].astype(jnp.int32)
    dst = edge_index[1].astype(jnp.int32)
    padi = jnp.full((EPAD - E,), N, jnp.int32)
    src2 = jnp.concatenate([src, padi]).reshape(ROWS, B)
    dst2 = jnp.concatenate([dst, padi]).reshape(ROWS, B)
    xp = jnp.pad(x.astype(jnp.bfloat16), ((0, NPAD - N), (0, 0)))
    xcs = [xp[:, t * DC:(t + 1) * DC] for t in range(2)]
    s1, s2, cnt = _sc_agg(src2, dst2, *xcs)
    bias = (b_self + (1.0 - ALPHA) * b1 + ALPHA * b2)[None, :]
    bf = jnp.bfloat16
    self_out = _self_mm(xcs[0], xcs[1], W_self.T.astype(bf), bias)
    return _combine(self_out, s1, s2, cnt, W1.T.astype(bf), W2.T.astype(bf))
